# baseline (speedup 1.0000x reference)
import jax
import jax.numpy as jnp
from jax.experimental import pallas as pl
from jax.experimental.pallas import tpu as pltpu

N_NODES = 10000
CH = 2000
NCH = N_NODES // CH


def _fused_mlp_kernel(x_hbm, W1_ref, b1_ref, W2_ref, b2_ref, W3_ref, b3_ref,
                      Wo1_ref, bo1_ref, Wo2_ref, bo2_ref, out_ref,
                      xbuf, sems, acc_ref):
    def copy(slot, idx):
        return pltpu.make_async_copy(
            x_hbm.at[pl.ds(idx * CH, CH), :], xbuf.at[slot], sems.at[slot])

    copy(0, 0).start()
    acc_ref[...] = jnp.zeros_like(acc_ref)
    for c in range(NCH):
        slot = c % 2
        if c + 1 < NCH:
            copy((c + 1) % 2, c + 1).start()
        copy(slot, c).wait()
        h = jnp.dot(xbuf[slot], W1_ref[...], preferred_element_type=jnp.float32)
        h = jnp.maximum(h + b1_ref[...], 0.0)
        h = jnp.dot(h, W2_ref[...], preferred_element_type=jnp.float32)
        h = jnp.maximum(h + b2_ref[...], 0.0)
        acc_ref[...] += jnp.sum(h, axis=0, keepdims=True)

    g = acc_ref[...] * (1.0 / N_NODES)
    g = jnp.dot(g, W3_ref[...], preferred_element_type=jnp.float32) + b3_ref[...]
    p = jnp.dot(g, Wo1_ref[...], preferred_element_type=jnp.float32)
    p = jnp.maximum(p + bo1_ref[...], 0.0)
    out_ref[...] = (jnp.dot(p, Wo2_ref[...], preferred_element_type=jnp.float32)
                    + bo2_ref[...])


def kernel(x, edge_index, W1, b1, W2, b2, W3, b3, Wo1, bo1, Wo2, bo2):
    del edge_index
    b1 = b1.reshape(1, -1)
    b2 = b2.reshape(1, -1)
    b3 = b3.reshape(1, -1)
    bo1 = bo1.reshape(1, -1)
    bo2 = bo2.reshape(1, -1)

    vmem = lambda a: pl.BlockSpec(a.shape, lambda: (0, 0))
    out = pl.pallas_call(
        _fused_mlp_kernel,
        in_specs=[
            pl.BlockSpec(memory_space=pl.ANY),
            vmem(W1), vmem(b1), vmem(W2), vmem(b2), vmem(W3), vmem(b3),
            vmem(Wo1), vmem(bo1), vmem(Wo2), vmem(bo2),
        ],
        out_specs=pl.BlockSpec((1, 1), lambda: (0, 0)),
        out_shape=jax.ShapeDtypeStruct((1, 1), jnp.float32),
        scratch_shapes=[
            pltpu.VMEM((2, CH, x.shape[1]), jnp.float32),
            pltpu.SemaphoreType.DMA((2,)),
            pltpu.VMEM((1, x.shape[1]), jnp.float32),
        ],
    )(x, W1, b1, W2, b2, W3, b3, Wo1, bo1, Wo2, bo2)
    return out
